# baseline (speedup 1.0000x reference)
import functools

import jax
import jax.numpy as jnp
from jax import lax
from jax.experimental import pallas as pl
from jax.experimental.pallas import tpu as pltpu
from jax.experimental.pallas import tpu_sc as plsc

_B = 8
_C = 2000
_QCOLS = 32
_OH0 = 12
_GCHUNK = 128
_GINNER = 16
_NW = 32
_LOG2E = 1.4426950408889634


def _sc_gather_call(n_tab, total_pad):
    per_w = total_pad // _NW
    rows_per_outer = _GINNER * _GCHUNK
    n_outer = per_w // rows_per_outer

    mesh = plsc.VectorSubcoreMesh(core_axis_name="c", subcore_axis_name="s")

    @functools.partial(
        pl.kernel,
        out_type=jax.ShapeDtypeStruct((total_pad, _QCOLS), jnp.bfloat16),
        mesh=mesh,
        scratch_types=[
            pltpu.VMEM((rows_per_outer,), jnp.int32),
            pltpu.VMEM((rows_per_outer, _QCOLS), jnp.bfloat16),
            pltpu.SemaphoreType.DMA,
        ],
        compiler_params=pltpu.CompilerParams(use_tc_tiling_on_sc=False),
    )
    def gather(tab_hbm, idx_hbm, out_hbm, idx_v, rows_v, sem):
        wid = lax.axis_index("s") * 2 + lax.axis_index("c")

        def outer(o, carry):
            base = wid * per_w + o * rows_per_outer
            pltpu.sync_copy(idx_hbm.at[pl.ds(base, rows_per_outer)], idx_v)
            cps = [
                pltpu.async_copy(
                    tab_hbm.at[idx_v.at[pl.ds(j * _GCHUNK, _GCHUNK)]],
                    rows_v.at[pl.ds(j * _GCHUNK, _GCHUNK)],
                    sem,
                )
                for j in range(_GINNER)
            ]
            for cp in cps:
                cp.wait()
            pltpu.sync_copy(rows_v, out_hbm.at[pl.ds(base, rows_per_outer)])
            return carry

        lax.fori_loop(0, n_outer, outer, 0)

    return gather


def _acc_call(nv, n_steps, st, row_offsets, sign=1, normalize=False,
              has_prev=False, interpret=False):
    def body(*refs):
        i = pl.program_id(0)
        g_refs = list(refs[:nv])
        vt_ref, lin_ref = refs[nv], refs[nv + 1]
        rest = list(refs[nv + 2:])
        if has_prev:
            prev_ref = rest.pop(0)
        out_ref, acc_v = rest

        @pl.when(i == 0)
        def _init():
            acc_v[...] = jnp.zeros_like(acc_v)

        h = None
        for r in g_refs:
            hr = jnp.dot(r[...], vt_ref[...], preferred_element_type=jnp.float32)
            h = hr if h is None else jnp.maximum(h, hr)
        sig = 1.0 / (1.0 + jnp.exp2(h - lin_ref[...]))
        part = lax.dot_general(
            g_refs[0][...], sig.astype(jnp.bfloat16),
            (((0,), (0,)), ((), ())),
            preferred_element_type=jnp.float32,
        )
        acc_v[...] += part

        @pl.when(i == n_steps - 1)
        def _emit():
            a = acc_v[_OH0 : _OH0 + _B, :]
            u = -a if sign < 0 else a
            if has_prev:
                u = prev_ref[...] + u
            if normalize:
                u = u / jnp.max(u, axis=1, keepdims=True)
            out_ref[...] = u

    in_specs = [
        pl.BlockSpec((_C, _QCOLS), lambda i, off=off: (i + off, 0))
        for off in row_offsets
    ]
    in_specs += [
        pl.BlockSpec((_QCOLS, st), lambda i: (0, 0)),
        pl.BlockSpec((1, st), lambda i: (0, 0)),
    ]
    if has_prev:
        in_specs += [pl.BlockSpec((_B, st), lambda i: (0, 0))]
    return pl.pallas_call(
        body,
        grid=(n_steps,),
        in_specs=in_specs,
        out_specs=pl.BlockSpec((_B, st), lambda i: (0, 0)),
        out_shape=jax.ShapeDtypeStruct((_B, st), jnp.float32),
        scratch_shapes=[pltpu.VMEM((_QCOLS, st), jnp.float32)],
        interpret=interpret,
    )


def _quantize(x, v, lin, batch, scale, d, t, s, st):
    sc = jnp.asarray(scale, jnp.float32) * _LOG2E
    xsf = x * sc
    xhi = xsf.astype(jnp.bfloat16)
    xlo = (xsf - xhi.astype(jnp.float32)).astype(jnp.bfloat16)
    oh = (batch[:, None] == jnp.arange(_B, dtype=batch.dtype)[None, :]).astype(
        jnp.bfloat16
    )
    xq = jnp.concatenate(
        [xhi, xlo, xhi, xlo, oh,
         jnp.zeros((x.shape[0], _QCOLS - _OH0 - _B), jnp.bfloat16)],
        axis=1,
    )
    vhi = v.astype(jnp.bfloat16)
    vlo = (v - vhi.astype(jnp.float32)).astype(jnp.bfloat16)
    vq = jnp.concatenate(
        [vhi, vhi, vlo, vlo, jnp.zeros((_QCOLS - 4 * d, t), jnp.bfloat16)], axis=0
    )
    vtq = jnp.tile(vq, (1, s))
    linr = (sc * jnp.repeat(lin.reshape(s).astype(jnp.float32), t)).reshape(1, st)
    return xq, vtq, linr


def kernel(x, v, lin, edge_index, face, triangulation, batch, index, scale):
    n, d = x.shape
    t = v.shape[1]
    s = lin.shape[0]
    e = edge_index.shape[1]
    f = face.shape[1]
    st = s * t

    xq, vtq, linr = _quantize(x, v, lin, batch, scale, d, t, s, st)

    allidx = jnp.concatenate(
        [edge_index[0], edge_index[1], face[0], face[1], face[2]]
    )
    total = 2 * e + 3 * f
    tp = (-total) % (_NW * _GINNER * _GCHUNK)
    allidx = jnp.concatenate([allidx, jnp.zeros((tp,), jnp.int32)])
    g = _sc_gather_call(n, total + tp)(xq, allidx)

    e_blk = e // _C
    f_blk = f // _C
    acc_n = _acc_call(1, n // _C, st, [0])(xq, vtq, linr)
    acc_e = _acc_call(2, e_blk, st, [0, e_blk], sign=-1, has_prev=True)(
        g, g, vtq, linr, acc_n
    )
    ect = _acc_call(
        3, f_blk, st, [2 * e_blk, 2 * e_blk + f_blk, 2 * e_blk + 2 * f_blk],
        normalize=True, has_prev=True,
    )(g, g, g, vtq, linr, acc_e)
    return ect.reshape(_B, s, t)


# speedup vs baseline: 1.0649x; 1.0649x over previous
import functools

import jax
import jax.numpy as jnp
from jax import lax
from jax.experimental import pallas as pl
from jax.experimental.pallas import tpu as pltpu
from jax.experimental.pallas import tpu_sc as plsc

_B = 8
_C = 2000
_QCOLS = 32
_OH0 = 12
_GCHUNK = 128
_GINNER = 16
_NW = 32
_LOG2E = 1.4426950408889634


def _sc_gather_call(n_tab, total_pad):
    per_w = total_pad // _NW
    rows_per_outer = _GINNER * _GCHUNK
    n_outer = per_w // rows_per_outer
    idx_rows_w = per_w // _GCHUNK

    mesh = plsc.VectorSubcoreMesh(core_axis_name="c", subcore_axis_name="s")

    @functools.partial(
        pl.kernel,
        out_type=jax.ShapeDtypeStruct((total_pad, _QCOLS), jnp.bfloat16),
        mesh=mesh,
        scratch_types=[
            pltpu.VMEM((_GINNER, _GCHUNK), jnp.int32),
            pltpu.VMEM((rows_per_outer, _QCOLS), jnp.bfloat16),
            pltpu.SemaphoreType.DMA,
        ],
        compiler_params=pltpu.CompilerParams(use_tc_tiling_on_sc=False),
    )
    def gather(tab_hbm, idx_hbm, out_hbm, idx_v, rows_v, sem):
        wid = lax.axis_index("s") * 2 + lax.axis_index("c")

        def outer(o, carry):
            pltpu.sync_copy(
                idx_hbm.at[pl.ds(wid * idx_rows_w + o * _GINNER, _GINNER)], idx_v
            )
            cps = [
                pltpu.async_copy(
                    tab_hbm.at[idx_v.at[j]],
                    rows_v.at[pl.ds(j * _GCHUNK, _GCHUNK)],
                    sem,
                )
                for j in range(_GINNER)
            ]
            for cp in cps:
                cp.wait()
            pltpu.sync_copy(
                rows_v,
                out_hbm.at[pl.ds(wid * per_w + o * rows_per_outer, rows_per_outer)],
            )
            return carry

        lax.fori_loop(0, n_outer, outer, 0)

    return gather


def _acc_call(nv, n_steps, st, row_offsets, sign=1, normalize=False,
              has_prev=False, interpret=False):
    def body(*refs):
        i = pl.program_id(0)
        g_refs = list(refs[:nv])
        vt_ref, lin_ref = refs[nv], refs[nv + 1]
        rest = list(refs[nv + 2:])
        if has_prev:
            prev_ref = rest.pop(0)
        out_ref, acc_v = rest

        @pl.when(i == 0)
        def _init():
            acc_v[...] = jnp.zeros_like(acc_v)

        h = None
        for r in g_refs:
            hr = jnp.dot(r[...], vt_ref[...], preferred_element_type=jnp.float32)
            h = hr if h is None else jnp.maximum(h, hr)
        sig = 1.0 / (1.0 + jnp.exp2(h - lin_ref[...]))
        part = lax.dot_general(
            g_refs[0][...], sig.astype(jnp.bfloat16),
            (((0,), (0,)), ((), ())),
            preferred_element_type=jnp.float32,
        )
        acc_v[...] += part

        @pl.when(i == n_steps - 1)
        def _emit():
            a = acc_v[_OH0 : _OH0 + _B, :]
            u = -a if sign < 0 else a
            if has_prev:
                u = prev_ref[...] + u
            if normalize:
                u = u / jnp.max(u, axis=1, keepdims=True)
            out_ref[...] = u

    in_specs = [
        pl.BlockSpec((_C, _QCOLS), lambda i, off=off: (i + off, 0))
        for off in row_offsets
    ]
    in_specs += [
        pl.BlockSpec((_QCOLS, st), lambda i: (0, 0)),
        pl.BlockSpec((1, st), lambda i: (0, 0)),
    ]
    if has_prev:
        in_specs += [pl.BlockSpec((_B, st), lambda i: (0, 0))]
    return pl.pallas_call(
        body,
        grid=(n_steps,),
        in_specs=in_specs,
        out_specs=pl.BlockSpec((_B, st), lambda i: (0, 0)),
        out_shape=jax.ShapeDtypeStruct((_B, st), jnp.float32),
        scratch_shapes=[pltpu.VMEM((_QCOLS, st), jnp.float32)],
        interpret=interpret,
    )


def _quantize(x, v, lin, batch, scale, d, t, s, st):
    sc = jnp.asarray(scale, jnp.float32) * _LOG2E
    xsf = x * sc
    xhi = xsf.astype(jnp.bfloat16)
    xlo = (xsf - xhi.astype(jnp.float32)).astype(jnp.bfloat16)
    oh = (batch[:, None] == jnp.arange(_B, dtype=batch.dtype)[None, :]).astype(
        jnp.bfloat16
    )
    xq = jnp.concatenate(
        [xhi, xlo, xhi, xlo, oh,
         jnp.zeros((x.shape[0], _QCOLS - _OH0 - _B), jnp.bfloat16)],
        axis=1,
    )
    vhi = v.astype(jnp.bfloat16)
    vlo = (v - vhi.astype(jnp.float32)).astype(jnp.bfloat16)
    vq = jnp.concatenate(
        [vhi, vhi, vlo, vlo, jnp.zeros((_QCOLS - 4 * d, t), jnp.bfloat16)], axis=0
    )
    vtq = jnp.tile(vq, (1, s))
    linr = (sc * jnp.repeat(lin.reshape(s).astype(jnp.float32), t)).reshape(1, st)
    return xq, vtq, linr


def kernel(x, v, lin, edge_index, face, triangulation, batch, index, scale):
    n, d = x.shape
    t = v.shape[1]
    s = lin.shape[0]
    e = edge_index.shape[1]
    f = face.shape[1]
    st = s * t

    xq, vtq, linr = _quantize(x, v, lin, batch, scale, d, t, s, st)

    align = _NW * _GINNER * _GCHUNK

    def gather_of(parts, total_rows):
        tp = (-total_rows) % align
        idx = jnp.concatenate(parts + [jnp.zeros((tp,), jnp.int32)])
        return _sc_gather_call(n, total_rows + tp)(xq, idx.reshape(-1, _GCHUNK))

    ge = gather_of([edge_index[0], edge_index[1]], 2 * e)
    gf = gather_of([face[0], face[1], face[2]], 3 * f)

    e_blk = e // _C
    f_blk = f // _C
    acc_n = _acc_call(1, n // _C, st, [0])(xq, vtq, linr)
    acc_e = _acc_call(2, e_blk, st, [0, e_blk], sign=-1, has_prev=True)(
        ge, ge, vtq, linr, acc_n
    )
    ect = _acc_call(
        3, f_blk, st, [0, f_blk, 2 * f_blk], normalize=True, has_prev=True,
    )(gf, gf, gf, vtq, linr, acc_e)
    return ect.reshape(_B, s, t)
